# baseline (speedup 1.0000x reference)
import functools

import jax
import jax.numpy as jnp
from jax import lax
from jax.experimental import pallas as pl
from jax.experimental.pallas import tpu as pltpu
from jax.experimental.pallas import tpu_sc as plsc

N = 10000
E = 15625
D_IN = 5120
H1 = 1680
H2 = 640

FC = 128
NCHUNK1 = 14
F1P = NCHUNK1 * FC
NCHUNK2 = 6
F2P = NCHUNK2 * FC
EPT = 1024
E_PAD = 16 * EPT
NHALF = 5120
NPASS = 2
NROWS = 5248
OUTR = NPASS * NHALF
ZR = 158


def _zero_rows(zerobuf, accum, r0, total):
    q, off = divmod(total, ZR)
    o = 0
    for _ in range(q):
        pltpu.sync_copy(zerobuf, accum.at[pl.ds(r0 + o, ZR)])
        o += ZR
    if off:
        pltpu.sync_copy(zerobuf.at[pl.ds(0, off)],
                        accum.at[pl.ds(r0 + o, off)])


@functools.lru_cache(maxsize=None)
def _make_sc_segsum(nchunk):
    mesh = plsc.VectorSubcoreMesh(
        core_axis_name="c", subcore_axis_name="s", num_cores=2, num_subcores=16)
    cpc = nchunk // 2

    out_type = jax.ShapeDtypeStruct((nchunk, OUTR, FC), jnp.float32)
    scratch = [
        pltpu.VMEM((8, 128), jnp.int32),
        pltpu.VMEM((8, 128), jnp.int32),
        pltpu.VMEM((8, 128), jnp.int32),
        pltpu.VMEM((256, FC), jnp.float32),
        pltpu.VMEM((ZR, FC), jnp.float32),
        pltpu.VMEM_SHARED((NROWS, FC), jnp.float32),
        pltpu.SemaphoreType.DMA,
        pltpu.VMEM((8, 128), jnp.int32),
    ]

    def body(plflat, src3, dst3, agg, *rest):
        (srcbuf, dstbuf, idxbuf, rowsbuf, zerobuf, accum, sem, dstb) = rest

        core = lax.axis_index("c")
        sub = lax.axis_index("s")

        pltpu.sync_copy(src3.at[sub], srcbuf)
        pltpu.sync_copy(dst3.at[sub], dstbuf)

        z16 = jnp.zeros((16,), jnp.float32)

        def zrow(i, carry):
            for g in range(FC // 16):
                zerobuf[i, pl.ds(g * 16, 16)] = z16
            return carry
        lax.fori_loop(0, ZR, zrow, 0)

        first = core * cpc

        def irow(j, carry):
            def ig(g, c2):
                sl = pl.ds(pl.multiple_of(g * 16, 16), 16)
                idxbuf[j, sl] = srcbuf[j, sl] * nchunk + first
                return c2
            return lax.fori_loop(0, 8, ig, carry)
        lax.fori_loop(0, 8, irow, 0)

        def make_local_dst(base):
            def drow(j, carry):
                def dg(g, c2):
                    sl = pl.ds(pl.multiple_of(g * 16, 16), 16)
                    v = dstbuf[j, sl] - base
                    dummy = jnp.full((16,), NHALF, jnp.int32)
                    dstb[j, sl] = jnp.where((v >= 0) & (v < NHALF), v, dummy)
                    return c2
                return lax.fori_loop(0, 8, dg, carry)
            lax.fori_loop(0, 8, drow, 0)

        def gather_scatter(db):
            pend = pltpu.async_copy(plflat.at[idxbuf.at[0]],
                                    rowsbuf.at[pl.ds(0, 128)], sem)
            for j in range(8):
                nxt = None
                if j < 7:
                    nxt = pltpu.async_copy(
                        plflat.at[idxbuf.at[j + 1]],
                        rowsbuf.at[pl.ds(((j + 1) % 2) * 128, 128)], sem)
                pend.wait()
                pltpu.sync_copy(rowsbuf.at[pl.ds((j % 2) * 128, 128)],
                                accum.at[db.at[j]], add=True)
                pend = nxt

        def chunk(k, carry):
            c = first + k
            for p in range(NPASS):
                make_local_dst(p * NHALF)
                _zero_rows(zerobuf, accum, sub * 328, 328)
                plsc.subcore_barrier()
                gather_scatter(dstb)
                plsc.subcore_barrier()
                pltpu.sync_copy(
                    accum.at[pl.ds(sub * 320, 320)],
                    agg.at[c, pl.ds(p * NHALF + sub * 320, 320)])
                plsc.subcore_barrier()

            def inc_row(j, c1):
                def inc_g(g, c2):
                    sl = pl.ds(pl.multiple_of(g * 16, 16), 16)
                    idxbuf[j, sl] = idxbuf[j, sl] + 1
                    return c2
                return lax.fori_loop(0, 8, inc_g, c1)
            lax.fori_loop(0, 8, inc_row, 0)
            return carry
        lax.fori_loop(0, cpc, chunk, 0)

    return pl.kernel(body, out_type=out_type, mesh=mesh, scratch_types=scratch)


def _mm_body(x_ref, w_ref, b_ref, out_ref, acc_ref):
    k = pl.program_id(2)

    @pl.when(k == 0)
    def _():
        acc_ref[...] = jnp.zeros_like(acc_ref)

    acc_ref[...] += jnp.dot(x_ref[...].astype(jnp.bfloat16),
                            w_ref[...].astype(jnp.bfloat16),
                            preferred_element_type=jnp.float32)

    @pl.when(k == pl.num_programs(2) - 1)
    def _():
        out_ref[...] = acc_ref[...] + b_ref[...]


def _matmul(x, w, b2d, Mb, Nb, Kb, out_cols=None):
    M, K = x.shape
    _, Nc = w.shape
    grid = (M // Mb, Nc // Nb, K // Kb)
    return pl.pallas_call(
        _mm_body,
        grid=grid,
        in_specs=[
            pl.BlockSpec((Mb, Kb), lambda i, j, k: (i, k)),
            pl.BlockSpec((Kb, Nb), lambda i, j, k: (k, j)),
            pl.BlockSpec((1, Nb), lambda i, j, k: (0, j)),
        ],
        out_specs=pl.BlockSpec((Mb, Nb), lambda i, j, k: (i, j)),
        out_shape=jax.ShapeDtypeStruct((M, out_cols or Nc), jnp.float32),
        scratch_shapes=[pltpu.VMEM((Mb, Nb), jnp.float32)],
        compiler_params=pltpu.CompilerParams(
            dimension_semantics=("parallel", "parallel", "arbitrary")),
    )(x, w, b2d)


MB1 = 1000


def _h1_body(agg_ref, cntc_ref, p1r_ref, sc_ref, sh_ref, out_ref):
    rc = 1.0 / jnp.maximum(cntc_ref[0, :, FC - 1:FC], 1.0)
    z = agg_ref[0] * rc + p1r_ref[...]
    out_ref[...] = jnp.maximum(z, 0.0) * sc_ref[...] + sh_ref[...]


def _h1_combine(agg1, p1r, scale2d, shift2d):
    grid = (N // MB1, NCHUNK1)
    return pl.pallas_call(
        _h1_body,
        grid=grid,
        in_specs=[
            pl.BlockSpec((1, MB1, FC), lambda i, j: (j, i, 0)),
            pl.BlockSpec((1, MB1, FC), lambda i, j: (NCHUNK1 - 1, i, 0)),
            pl.BlockSpec((MB1, FC), lambda i, j: (i, j)),
            pl.BlockSpec((1, FC), lambda i, j: (0, j)),
            pl.BlockSpec((1, FC), lambda i, j: (0, j)),
        ],
        out_specs=pl.BlockSpec((MB1, FC), lambda i, j: (i, j)),
        out_shape=jax.ShapeDtypeStruct((N, F1P), jnp.float32),
    )(agg1, agg1, p1r, scale2d, shift2d)


MBT = 2000


def _tail_body(agg_ref, p2r_ref, w3, b3, w4, b4, w5, b5,
               out_ref, h2_ref):
    rc = 1.0 / jnp.maximum(agg_ref[NCHUNK2 - 1][:, FC - 1:FC], 1.0)
    for c in range(NCHUNK2):
        z = agg_ref[c] * rc + p2r_ref[:, c * FC:(c + 1) * FC]
        h2_ref[:, c * FC:(c + 1) * FC] = jnp.maximum(z, 0.0)
    a = jnp.maximum(jnp.dot(h2_ref[...], w3[...],
                            preferred_element_type=jnp.float32) + b3[...], 0.0)
    a = jnp.maximum(jnp.dot(a, w4[...],
                            preferred_element_type=jnp.float32) + b4[...], 0.0)
    out_ref[...] = jnp.dot(a, w5[...],
                           preferred_element_type=jnp.float32) + b5[...]


def _tail(agg2, p2r, W3p, b3, W4, b4, W5p, b5p):
    grid = (N // MBT,)
    return pl.pallas_call(
        _tail_body,
        grid=grid,
        in_specs=[
            pl.BlockSpec((NCHUNK2, MBT, FC), lambda i: (0, i, 0)),
            pl.BlockSpec((MBT, F2P), lambda i: (i, 0)),
            pl.BlockSpec(W3p.shape, lambda i: (0, 0)),
            pl.BlockSpec((1, 320), lambda i: (0, 0)),
            pl.BlockSpec(W4.shape, lambda i: (0, 0)),
            pl.BlockSpec((1, 160), lambda i: (0, 0)),
            pl.BlockSpec(W5p.shape, lambda i: (0, 0)),
            pl.BlockSpec((1, 128), lambda i: (0, 0)),
        ],
        out_specs=pl.BlockSpec((MBT, 128), lambda i: (i, 0)),
        out_shape=jax.ShapeDtypeStruct((N, 128), jnp.float32),
        scratch_shapes=[pltpu.VMEM((MBT, F2P), jnp.float32)],
    )(agg2, p2r, W3p, b3, W4, b4, W5p, b5p)


def _forward(x, edge_index, W1_l, W1_r, b1, W2_l, W2_r, b2,
             bn_gamma, bn_beta, bn_mean, bn_var, W3, b3, W4, b4, W5, b5,
             seg1, seg2):
    src = edge_index[0]
    dst = edge_index[1]
    pad = E_PAD - E
    src_p = jnp.concatenate(
        [src, jnp.zeros((pad,), jnp.int32)]).reshape(16, 8, 128)
    dst_p = jnp.concatenate(
        [dst, jnp.full((pad,), N, jnp.int32)]).reshape(16, 8, 128)

    W1l_p = jnp.pad(W1_l, ((0, 0), (0, F1P - H1)))
    W1r_p = jnp.pad(W1_r, ((0, 0), (0, F1P - H1)))
    b1_p = jnp.pad(b1, (0, F1P - H1)).reshape(1, F1P)
    ones_col1 = jnp.zeros((1, F1P), jnp.float32).at[0, F1P - 1].set(1.0)
    P1l = _matmul(x, W1l_p, ones_col1, Mb=1000, Nb=F1P, Kb=1024)
    agg1 = seg1(P1l.reshape(N * NCHUNK1, FC), src_p, dst_p)
    P1r = _matmul(x, W1r_p, b1_p, Mb=1000, Nb=F1P, Kb=1024)

    scale = jnp.pad(bn_gamma * lax.rsqrt(bn_var + 1e-5), (0, F1P - H1))
    shift = jnp.pad(bn_beta - bn_mean * bn_gamma * lax.rsqrt(bn_var + 1e-5),
                    (0, F1P - H1))
    h1 = _h1_combine(agg1, P1r, scale.reshape(1, F1P), shift.reshape(1, F1P))

    W2l_p = jnp.pad(W2_l, ((0, F1P - H1), (0, F2P - H2)))
    W2r_p = jnp.pad(W2_r, ((0, F1P - H1), (0, F2P - H2)))
    b2_p = jnp.pad(b2, (0, F2P - H2)).reshape(1, F2P)
    ones_col2 = jnp.zeros((1, F2P), jnp.float32).at[0, F2P - 1].set(1.0)
    P2l = _matmul(h1, W2l_p, ones_col2, Mb=1000, Nb=F2P, Kb=F1P)
    agg2 = seg2(P2l.reshape(N * NCHUNK2, FC), src_p, dst_p)
    P2r = _matmul(h1, W2r_p, b2_p, Mb=1000, Nb=F2P, Kb=F1P)

    W3p = jnp.pad(W3, ((0, F2P - H2), (0, 0)))
    W5p = jnp.pad(W5, ((0, 0), (0, 126)))
    b5p = jnp.pad(b5, (0, 126)).reshape(1, 128)
    out128 = _tail(agg2, P2r, W3p, b3.reshape(1, 320),
                   W4, b4.reshape(1, 160), W5p, b5p)
    return out128[:, :2]


def kernel(x, edge_index, W1_l, W1_r, b1, W2_l, W2_r, b2,
           bn_gamma, bn_beta, bn_mean, bn_var, W3, b3, W4, b4, W5, b5):
    return _forward(x, edge_index, W1_l, W1_r, b1, W2_l, W2_r, b2,
                    bn_gamma, bn_beta, bn_mean, bn_var, W3, b3, W4, b4, W5, b5,
                    _make_sc_segsum(NCHUNK1),
                    _make_sc_segsum(NCHUNK2))


# speedup vs baseline: 1.0869x; 1.0869x over previous
import functools

import jax
import jax.numpy as jnp
from jax import lax
from jax.experimental import pallas as pl
from jax.experimental.pallas import tpu as pltpu
from jax.experimental.pallas import tpu_sc as plsc

N = 10000
E = 15625
D_IN = 5120
H1 = 1680
H2 = 640

FC = 128
NCHUNK1 = 14
F1P = NCHUNK1 * FC
NCHUNK2 = 6
F2P = NCHUNK2 * FC
EPT = 1024
E_PAD = 16 * EPT
NROWS = 10240
OUTR = NROWS
ZR = 64


def _zero_rows(zerobuf, accum, r0, total):
    q, off = divmod(total, ZR)
    o = 0
    for _ in range(q):
        pltpu.sync_copy(zerobuf, accum.at[pl.ds(r0 + o, ZR)])
        o += ZR
    if off:
        pltpu.sync_copy(zerobuf.at[pl.ds(0, off)],
                        accum.at[pl.ds(r0 + o, off)])


@functools.lru_cache(maxsize=None)
def _make_sc_segsum(nchunk):
    mesh = plsc.VectorSubcoreMesh(
        core_axis_name="c", subcore_axis_name="s", num_cores=2, num_subcores=16)
    cpc = nchunk // 2

    out_type = jax.ShapeDtypeStruct((nchunk, OUTR, FC), jnp.float32)
    scratch = [
        pltpu.VMEM((8, 128), jnp.int32),
        pltpu.VMEM((8, 128), jnp.int32),
        pltpu.VMEM((8, 128), jnp.int32),
        pltpu.VMEM((256, FC), jnp.float32),
        pltpu.VMEM((ZR, FC), jnp.float32),
        pltpu.VMEM_SHARED((NROWS, FC), jnp.float32),
        pltpu.SemaphoreType.DMA,
    ]

    def body(plflat, src3, dst3, agg, *rest):
        (srcbuf, dstbuf, idxbuf, rowsbuf, zerobuf, accum, sem) = rest

        core = lax.axis_index("c")
        sub = lax.axis_index("s")

        pltpu.sync_copy(src3.at[sub], srcbuf)
        pltpu.sync_copy(dst3.at[sub], dstbuf)

        z16 = jnp.zeros((16,), jnp.float32)

        def zrow(i, carry):
            for g in range(FC // 16):
                zerobuf[i, pl.ds(g * 16, 16)] = z16
            return carry
        lax.fori_loop(0, ZR, zrow, 0)

        first = core * cpc

        def irow(j, carry):
            def ig(g, c2):
                sl = pl.ds(pl.multiple_of(g * 16, 16), 16)
                idxbuf[j, sl] = srcbuf[j, sl] * nchunk + first
                return c2
            return lax.fori_loop(0, 8, ig, carry)
        lax.fori_loop(0, 8, irow, 0)

        def gather_scatter():
            pend = pltpu.async_copy(plflat.at[idxbuf.at[0]],
                                    rowsbuf.at[pl.ds(0, 128)], sem)
            for j in range(8):
                nxt = None
                if j < 7:
                    nxt = pltpu.async_copy(
                        plflat.at[idxbuf.at[j + 1]],
                        rowsbuf.at[pl.ds(((j + 1) % 2) * 128, 128)], sem)
                pend.wait()
                pltpu.sync_copy(rowsbuf.at[pl.ds((j % 2) * 128, 128)],
                                accum.at[dstbuf.at[j]], add=True)
                pend = nxt

        def chunk(k, carry):
            c = first + k
            _zero_rows(zerobuf, accum, sub * 640, 640)
            plsc.subcore_barrier()
            gather_scatter()
            plsc.subcore_barrier()
            pltpu.sync_copy(accum.at[pl.ds(sub * 640, 640)],
                            agg.at[c, pl.ds(sub * 640, 640)])
            plsc.subcore_barrier()

            def inc_row(j, c1):
                def inc_g(g, c2):
                    sl = pl.ds(pl.multiple_of(g * 16, 16), 16)
                    idxbuf[j, sl] = idxbuf[j, sl] + 1
                    return c2
                return lax.fori_loop(0, 8, inc_g, c1)
            lax.fori_loop(0, 8, inc_row, 0)
            return carry
        lax.fori_loop(0, cpc, chunk, 0)

    return pl.kernel(body, out_type=out_type, mesh=mesh, scratch_types=scratch)


def _mm_body(x_ref, w_ref, b_ref, out_ref, acc_ref):
    k = pl.program_id(2)

    @pl.when(k == 0)
    def _():
        acc_ref[...] = jnp.zeros_like(acc_ref)

    acc_ref[...] += jnp.dot(x_ref[...].astype(jnp.bfloat16),
                            w_ref[...].astype(jnp.bfloat16),
                            preferred_element_type=jnp.float32)

    @pl.when(k == pl.num_programs(2) - 1)
    def _():
        out_ref[...] = (acc_ref[...] + b_ref[...]).astype(out_ref.dtype)


def _matmul(x, w, b2d, Mb, Nb, Kb, out_cols=None, out_dtype=jnp.float32):
    M, K = x.shape
    _, Nc = w.shape
    grid = (M // Mb, Nc // Nb, K // Kb)
    return pl.pallas_call(
        _mm_body,
        grid=grid,
        in_specs=[
            pl.BlockSpec((Mb, Kb), lambda i, j, k: (i, k)),
            pl.BlockSpec((Kb, Nb), lambda i, j, k: (k, j)),
            pl.BlockSpec((1, Nb), lambda i, j, k: (0, j)),
        ],
        out_specs=pl.BlockSpec((Mb, Nb), lambda i, j, k: (i, j)),
        out_shape=jax.ShapeDtypeStruct((M, out_cols or Nc), out_dtype),
        scratch_shapes=[pltpu.VMEM((Mb, Nb), jnp.float32)],
        compiler_params=pltpu.CompilerParams(
            dimension_semantics=("parallel", "parallel", "arbitrary")),
    )(x, w, b2d)


MB1 = 1000


def _h1_body(agg_ref, cntc_ref, p1r_ref, sc_ref, sh_ref, out_ref):
    cnt = cntc_ref[0, :, FC - 1:FC].astype(jnp.float32)
    rc = 1.0 / jnp.maximum(cnt, 1.0)
    z = agg_ref[0].astype(jnp.float32) * rc + p1r_ref[...]
    out_ref[...] = jnp.maximum(z, 0.0) * sc_ref[...] + sh_ref[...]


def _h1_combine(agg1, p1r, scale2d, shift2d):
    grid = (N // MB1, NCHUNK1)
    return pl.pallas_call(
        _h1_body,
        grid=grid,
        in_specs=[
            pl.BlockSpec((1, MB1, FC), lambda i, j: (j, i, 0)),
            pl.BlockSpec((1, MB1, FC), lambda i, j: (NCHUNK1 - 1, i, 0)),
            pl.BlockSpec((MB1, FC), lambda i, j: (i, j)),
            pl.BlockSpec((1, FC), lambda i, j: (0, j)),
            pl.BlockSpec((1, FC), lambda i, j: (0, j)),
        ],
        out_specs=pl.BlockSpec((MB1, FC), lambda i, j: (i, j)),
        out_shape=jax.ShapeDtypeStruct((N, F1P), jnp.float32),
    )(agg1, agg1, p1r, scale2d, shift2d)


MBT = 2000


def _tail_body(agg_ref, p2r_ref, w3, b3, w4, b4, w5, b5,
               out_ref, h2_ref):
    cnt = agg_ref[NCHUNK2 - 1][:, FC - 1:FC].astype(jnp.float32)
    rc = 1.0 / jnp.maximum(cnt, 1.0)
    for c in range(NCHUNK2):
        z = agg_ref[c].astype(jnp.float32) * rc + p2r_ref[:, c * FC:(c + 1) * FC]
        h2_ref[:, c * FC:(c + 1) * FC] = jnp.maximum(z, 0.0)
    a = jnp.maximum(jnp.dot(h2_ref[...], w3[...],
                            preferred_element_type=jnp.float32) + b3[...], 0.0)
    a = jnp.maximum(jnp.dot(a, w4[...],
                            preferred_element_type=jnp.float32) + b4[...], 0.0)
    out_ref[...] = jnp.dot(a, w5[...],
                           preferred_element_type=jnp.float32) + b5[...]


def _tail(agg2, p2r, W3p, b3, W4, b4, W5p, b5p):
    grid = (N // MBT,)
    return pl.pallas_call(
        _tail_body,
        grid=grid,
        in_specs=[
            pl.BlockSpec((NCHUNK2, MBT, FC), lambda i: (0, i, 0)),
            pl.BlockSpec((MBT, F2P), lambda i: (i, 0)),
            pl.BlockSpec(W3p.shape, lambda i: (0, 0)),
            pl.BlockSpec((1, 320), lambda i: (0, 0)),
            pl.BlockSpec(W4.shape, lambda i: (0, 0)),
            pl.BlockSpec((1, 160), lambda i: (0, 0)),
            pl.BlockSpec(W5p.shape, lambda i: (0, 0)),
            pl.BlockSpec((1, 128), lambda i: (0, 0)),
        ],
        out_specs=pl.BlockSpec((MBT, 128), lambda i: (i, 0)),
        out_shape=jax.ShapeDtypeStruct((N, 128), jnp.float32),
        scratch_shapes=[pltpu.VMEM((MBT, F2P), jnp.float32)],
    )(agg2, p2r, W3p, b3, W4, b4, W5p, b5p)


def _forward(x, edge_index, W1_l, W1_r, b1, W2_l, W2_r, b2,
             bn_gamma, bn_beta, bn_mean, bn_var, W3, b3, W4, b4, W5, b5,
             seg1, seg2):
    src = edge_index[0]
    dst = edge_index[1]
    pad = E_PAD - E
    src_p = jnp.concatenate(
        [src, jnp.zeros((pad,), jnp.int32)]).reshape(16, 8, 128)
    dst_p = jnp.concatenate(
        [dst, jnp.full((pad,), N, jnp.int32)]).reshape(16, 8, 128)

    W1l_p = jnp.pad(W1_l, ((0, 0), (0, F1P - H1)))
    W1r_p = jnp.pad(W1_r, ((0, 0), (0, F1P - H1)))
    b1_p = jnp.pad(b1, (0, F1P - H1)).reshape(1, F1P)
    ones_col1 = jnp.zeros((1, F1P), jnp.float32).at[0, F1P - 1].set(1.0)
    P1l = _matmul(x, W1l_p, ones_col1, Mb=1000, Nb=F1P, Kb=1024)
    agg1 = seg1(P1l.reshape(N * NCHUNK1, FC), src_p, dst_p)
    P1r = _matmul(x, W1r_p, b1_p, Mb=1000, Nb=F1P, Kb=1024)

    scale = jnp.pad(bn_gamma * lax.rsqrt(bn_var + 1e-5), (0, F1P - H1))
    shift = jnp.pad(bn_beta - bn_mean * bn_gamma * lax.rsqrt(bn_var + 1e-5),
                    (0, F1P - H1))
    h1 = _h1_combine(agg1, P1r, scale.reshape(1, F1P), shift.reshape(1, F1P))

    W2l_p = jnp.pad(W2_l, ((0, F1P - H1), (0, F1P - H2)))
    W2r_p = jnp.pad(W2_r, ((0, F1P - H1), (0, F2P - H2)))
    b2_p = jnp.pad(b2, (0, F2P - H2)).reshape(1, F2P)
    ones_col2 = jnp.zeros((1, F1P), jnp.float32).at[0, F2P - 1].set(1.0)
    P2l = _matmul(h1, W2l_p, ones_col2, Mb=1000, Nb=F1P, Kb=F1P)
    agg2 = seg2(P2l.reshape(N * NCHUNK1, FC), src_p, dst_p)
    P2r = _matmul(h1, W2r_p, b2_p, Mb=1000, Nb=F2P, Kb=F1P)

    W3p = jnp.pad(W3, ((0, F2P - H2), (0, 0)))
    W5p = jnp.pad(W5, ((0, 0), (0, 126)))
    b5p = jnp.pad(b5, (0, 126)).reshape(1, 128)
    out128 = _tail(agg2, P2r, W3p, b3.reshape(1, 320),
                   W4, b4.reshape(1, 160), W5p, b5p)
    return out128[:, :2]


def kernel(x, edge_index, W1_l, W1_r, b1, W2_l, W2_r, b2,
           bn_gamma, bn_beta, bn_mean, bn_var, W3, b3, W4, b4, W5, b5):
    return _forward(x, edge_index, W1_l, W1_r, b1, W2_l, W2_r, b2,
                    bn_gamma, bn_beta, bn_mean, bn_var, W3, b3, W4, b4, W5, b5,
                    _make_sc_segsum(NCHUNK1),
                    _make_sc_segsum(NCHUNK1))


# speedup vs baseline: 1.0939x; 1.0064x over previous
import functools

import jax
import jax.numpy as jnp
from jax import lax
from jax.experimental import pallas as pl
from jax.experimental.pallas import tpu as pltpu
from jax.experimental.pallas import tpu_sc as plsc

N = 10000
E = 15625
D_IN = 5120
H1 = 1680
H2 = 640

FC = 128
NCHUNK1 = 14
F1P = NCHUNK1 * FC
NCHUNK2 = 6
F2P = NCHUNK2 * FC
EPT = 1024
E_PAD = 16 * EPT
NROWS = 10240
OUTR = NROWS
ZR = 64


def _zero_rows(zerobuf, accum, r0, total):
    q, off = divmod(total, ZR)
    o = 0
    for _ in range(q):
        pltpu.sync_copy(zerobuf, accum.at[pl.ds(r0 + o, ZR)])
        o += ZR
    if off:
        pltpu.sync_copy(zerobuf.at[pl.ds(0, off)],
                        accum.at[pl.ds(r0 + o, off)])


@functools.lru_cache(maxsize=None)
def _make_sc_segsum(nchunk):
    mesh = plsc.VectorSubcoreMesh(
        core_axis_name="c", subcore_axis_name="s", num_cores=2, num_subcores=16)

    out_type = jax.ShapeDtypeStruct((nchunk, OUTR, FC), jnp.float32)
    scratch = [
        pltpu.VMEM((8, 128), jnp.int32),
        pltpu.VMEM((8, 128), jnp.int32),
        pltpu.VMEM((8, 128), jnp.int32),
        pltpu.VMEM((256, FC), jnp.float32),
        pltpu.VMEM((ZR, FC), jnp.float32),
        pltpu.VMEM_SHARED((NROWS, FC), jnp.float32),
        pltpu.SemaphoreType.DMA,
    ]

    def body(plflat, src3, dst3, agg, *rest):
        (srcbuf, dstbuf, idxbuf, rowsbuf, zerobuf, accum, sem) = rest

        core = lax.axis_index("c")
        sub = lax.axis_index("s")

        pltpu.sync_copy(src3.at[sub], srcbuf)
        pltpu.sync_copy(dst3.at[sub], dstbuf)

        z16 = jnp.zeros((16,), jnp.float32)

        def zrow(i, carry):
            for g in range(FC // 16):
                zerobuf[i, pl.ds(g * 16, 16)] = z16
            return carry
        lax.fori_loop(0, ZR, zrow, 0)

        def irow(j, carry):
            def ig(g, c2):
                sl = pl.ds(pl.multiple_of(g * 16, 16), 16)
                idxbuf[j, sl] = srcbuf[j, sl] * nchunk + core
                return c2
            return lax.fori_loop(0, 8, ig, carry)
        lax.fori_loop(0, 8, irow, 0)

        def gather_scatter():
            pend = pltpu.async_copy(plflat.at[idxbuf.at[0]],
                                    rowsbuf.at[pl.ds(0, 128)], sem)
            for j in range(8):
                nxt = None
                if j < 7:
                    nxt = pltpu.async_copy(
                        plflat.at[idxbuf.at[j + 1]],
                        rowsbuf.at[pl.ds(((j + 1) % 2) * 128, 128)], sem)
                pend.wait()
                pltpu.sync_copy(rowsbuf.at[pl.ds((j % 2) * 128, 128)],
                                accum.at[dstbuf.at[j]], add=True)
                pend = nxt

        def chunk(k, carry):
            c = core + 2 * k
            _zero_rows(zerobuf, accum, sub * 640, 640)
            plsc.subcore_barrier()
            gather_scatter()
            plsc.subcore_barrier()
            pltpu.sync_copy(accum.at[pl.ds(sub * 640, 640)],
                            agg.at[c, pl.ds(sub * 640, 640)])
            plsc.subcore_barrier()

            def inc_row(j, c1):
                def inc_g(g, c2):
                    sl = pl.ds(pl.multiple_of(g * 16, 16), 16)
                    idxbuf[j, sl] = idxbuf[j, sl] + 2
                    return c2
                return lax.fori_loop(0, 8, inc_g, c1)
            lax.fori_loop(0, 8, inc_row, 0)
            return carry
        lax.fori_loop(0, nchunk // 2, chunk, 0)

    return pl.kernel(body, out_type=out_type, mesh=mesh, scratch_types=scratch)


def _mm_body(x_ref, w_ref, b_ref, out_ref, acc_ref):
    k = pl.program_id(2)

    @pl.when(k == 0)
    def _():
        acc_ref[...] = jnp.zeros_like(acc_ref)

    acc_ref[...] += jnp.dot(x_ref[...].astype(jnp.bfloat16),
                            w_ref[...].astype(jnp.bfloat16),
                            preferred_element_type=jnp.float32)

    @pl.when(k == pl.num_programs(2) - 1)
    def _():
        out_ref[...] = (acc_ref[...] + b_ref[...]).astype(out_ref.dtype)


def _matmul(x, w, b2d, Mb, Nb, Kb, out_cols=None, out_dtype=jnp.float32):
    M, K = x.shape
    _, Nc = w.shape
    grid = (M // Mb, Nc // Nb, K // Kb)
    return pl.pallas_call(
        _mm_body,
        grid=grid,
        in_specs=[
            pl.BlockSpec((Mb, Kb), lambda i, j, k: (i, k)),
            pl.BlockSpec((Kb, Nb), lambda i, j, k: (k, j)),
            pl.BlockSpec((1, Nb), lambda i, j, k: (0, j)),
        ],
        out_specs=pl.BlockSpec((Mb, Nb), lambda i, j, k: (i, j)),
        out_shape=jax.ShapeDtypeStruct((M, out_cols or Nc), out_dtype),
        scratch_shapes=[pltpu.VMEM((Mb, Nb), jnp.float32)],
        compiler_params=pltpu.CompilerParams(
            dimension_semantics=("parallel", "parallel", "arbitrary")),
    )(x, w, b2d)


MB1 = 1000


def _h1_body(agg_ref, cntc_ref, p1r_ref, sc_ref, sh_ref, out_ref):
    cnt = cntc_ref[0, :, FC - 1:FC].astype(jnp.float32)
    rc = 1.0 / jnp.maximum(cnt, 1.0)
    z = agg_ref[0].astype(jnp.float32) * rc + p1r_ref[...]
    out_ref[...] = jnp.maximum(z, 0.0) * sc_ref[...] + sh_ref[...]


def _h1_combine(agg1, p1r, scale2d, shift2d):
    grid = (N // MB1, NCHUNK1)
    return pl.pallas_call(
        _h1_body,
        grid=grid,
        in_specs=[
            pl.BlockSpec((1, MB1, FC), lambda i, j: (j, i, 0)),
            pl.BlockSpec((1, MB1, FC), lambda i, j: (NCHUNK1 - 1, i, 0)),
            pl.BlockSpec((MB1, FC), lambda i, j: (i, j)),
            pl.BlockSpec((1, FC), lambda i, j: (0, j)),
            pl.BlockSpec((1, FC), lambda i, j: (0, j)),
        ],
        out_specs=pl.BlockSpec((MB1, FC), lambda i, j: (i, j)),
        out_shape=jax.ShapeDtypeStruct((N, F1P), jnp.float32),
    )(agg1, agg1, p1r, scale2d, shift2d)


MBT = 2000


def _tail_body(agg_ref, p2r_ref, w3, b3, w4, b4, w5, b5,
               out_ref, h2_ref):
    cnt = agg_ref[NCHUNK2 - 1][:, FC - 1:FC].astype(jnp.float32)
    rc = 1.0 / jnp.maximum(cnt, 1.0)
    for c in range(NCHUNK2):
        z = agg_ref[c].astype(jnp.float32) * rc + p2r_ref[:, c * FC:(c + 1) * FC]
        h2_ref[:, c * FC:(c + 1) * FC] = jnp.maximum(z, 0.0)
    a = jnp.maximum(jnp.dot(h2_ref[...], w3[...],
                            preferred_element_type=jnp.float32) + b3[...], 0.0)
    a = jnp.maximum(jnp.dot(a, w4[...],
                            preferred_element_type=jnp.float32) + b4[...], 0.0)
    out_ref[...] = jnp.dot(a, w5[...],
                           preferred_element_type=jnp.float32) + b5[...]


def _tail(agg2, p2r, W3p, b3, W4, b4, W5p, b5p):
    grid = (N // MBT,)
    return pl.pallas_call(
        _tail_body,
        grid=grid,
        in_specs=[
            pl.BlockSpec((NCHUNK2, MBT, FC), lambda i: (0, i, 0)),
            pl.BlockSpec((MBT, F2P), lambda i: (i, 0)),
            pl.BlockSpec(W3p.shape, lambda i: (0, 0)),
            pl.BlockSpec((1, 320), lambda i: (0, 0)),
            pl.BlockSpec(W4.shape, lambda i: (0, 0)),
            pl.BlockSpec((1, 160), lambda i: (0, 0)),
            pl.BlockSpec(W5p.shape, lambda i: (0, 0)),
            pl.BlockSpec((1, 128), lambda i: (0, 0)),
        ],
        out_specs=pl.BlockSpec((MBT, 128), lambda i: (i, 0)),
        out_shape=jax.ShapeDtypeStruct((N, 128), jnp.float32),
        scratch_shapes=[pltpu.VMEM((MBT, F2P), jnp.float32)],
    )(agg2, p2r, W3p, b3, W4, b4, W5p, b5p)


def _forward(x, edge_index, W1_l, W1_r, b1, W2_l, W2_r, b2,
             bn_gamma, bn_beta, bn_mean, bn_var, W3, b3, W4, b4, W5, b5,
             seg1, seg2):
    src = edge_index[0]
    dst = edge_index[1]
    pad = E_PAD - E
    src_p = jnp.concatenate(
        [src, jnp.zeros((pad,), jnp.int32)]).reshape(16, 8, 128)
    dst_p = jnp.concatenate(
        [dst, jnp.full((pad,), N, jnp.int32)]).reshape(16, 8, 128)

    W1l_p = jnp.pad(W1_l, ((0, 0), (0, F1P - H1)))
    W1r_p = jnp.pad(W1_r, ((0, 0), (0, F1P - H1)))
    b1_p = jnp.pad(b1, (0, F1P - H1)).reshape(1, F1P)
    ones_col1 = jnp.zeros((1, F1P), jnp.float32).at[0, F1P - 1].set(1.0)
    P1l = _matmul(x, W1l_p, ones_col1, Mb=1000, Nb=F1P, Kb=1024)
    agg1 = seg1(P1l.reshape(N * NCHUNK1, FC), src_p, dst_p)
    P1r = _matmul(x, W1r_p, b1_p, Mb=1000, Nb=F1P, Kb=1024)

    scale = jnp.pad(bn_gamma * lax.rsqrt(bn_var + 1e-5), (0, F1P - H1))
    shift = jnp.pad(bn_beta - bn_mean * bn_gamma * lax.rsqrt(bn_var + 1e-5),
                    (0, F1P - H1))
    h1 = _h1_combine(agg1, P1r, scale.reshape(1, F1P), shift.reshape(1, F1P))

    W2l_p = jnp.pad(W2_l, ((0, F1P - H1), (0, F1P - H2)))
    W2r_p = jnp.pad(W2_r, ((0, F1P - H1), (0, F2P - H2)))
    b2_p = jnp.pad(b2, (0, F2P - H2)).reshape(1, F2P)
    ones_col2 = jnp.zeros((1, F1P), jnp.float32).at[0, F2P - 1].set(1.0)
    P2l = _matmul(h1, W2l_p, ones_col2, Mb=1000, Nb=F1P, Kb=F1P)
    agg2 = seg2(P2l.reshape(N * NCHUNK1, FC), src_p, dst_p)
    P2r = _matmul(h1, W2r_p, b2_p, Mb=1000, Nb=F2P, Kb=F1P)

    W3p = jnp.pad(W3, ((0, F2P - H2), (0, 0)))
    W5p = jnp.pad(W5, ((0, 0), (0, 126)))
    b5p = jnp.pad(b5, (0, 126)).reshape(1, 128)
    out128 = _tail(agg2, P2r, W3p, b3.reshape(1, 320),
                   W4, b4.reshape(1, 160), W5p, b5p)
    return out128[:, :2]


def kernel(x, edge_index, W1_l, W1_r, b1, W2_l, W2_r, b2,
           bn_gamma, bn_beta, bn_mean, bn_var, W3, b3, W4, b4, W5, b5):
    return _forward(x, edge_index, W1_l, W1_r, b1, W2_l, W2_r, b2,
                    bn_gamma, bn_beta, bn_mean, bn_var, W3, b3, W4, b4, W5, b5,
                    _make_sc_segsum(NCHUNK1),
                    _make_sc_segsum(NCHUNK1))


# speedup vs baseline: 1.2610x; 1.1528x over previous
import functools

import jax
import jax.numpy as jnp
from jax import lax
from jax.experimental import pallas as pl
from jax.experimental.pallas import tpu as pltpu
from jax.experimental.pallas import tpu_sc as plsc

N = 10000
E = 15625
D_IN = 5120
H1 = 1680
H2 = 640

FC = 128
NCHUNK1 = 14
F1P = NCHUNK1 * FC
NCHUNK2 = 6
F2P = NCHUNK2 * FC
EPT = 1024
E_PAD = 16 * EPT
NROWS = 10240
OUTR = NROWS
ZR = 64


def _zero_rows(zerobuf, accum, r0, total):
    q, off = divmod(total, ZR)
    o = 0
    for _ in range(q):
        pltpu.sync_copy(zerobuf, accum.at[pl.ds(r0 + o, ZR)])
        o += ZR
    if off:
        pltpu.sync_copy(zerobuf.at[pl.ds(0, off)],
                        accum.at[pl.ds(r0 + o, off)])


@functools.lru_cache(maxsize=None)
def _make_sc_segsum(nchunk, nchunk_out):
    mesh = plsc.VectorSubcoreMesh(
        core_axis_name="c", subcore_axis_name="s", num_cores=2, num_subcores=16)

    out_type = jax.ShapeDtypeStruct((nchunk_out, OUTR, FC), jnp.float32)
    scratch = [
        pltpu.VMEM((8, 128), jnp.int32),
        pltpu.VMEM((8, 128), jnp.int32),
        pltpu.VMEM((8, 128), jnp.int32),
        pltpu.VMEM((256, FC), jnp.float32),
        pltpu.VMEM((ZR, FC), jnp.float32),
        pltpu.VMEM_SHARED((NROWS, FC), jnp.float32),
        pltpu.SemaphoreType.DMA,
    ]

    def body(plflat, src3, dst3, agg, *rest):
        (srcbuf, dstbuf, idxbuf, rowsbuf, zerobuf, accum, sem) = rest

        core = lax.axis_index("c")
        sub = lax.axis_index("s")

        pltpu.sync_copy(src3.at[sub], srcbuf)
        pltpu.sync_copy(dst3.at[sub], dstbuf)

        z16 = jnp.zeros((16,), jnp.float32)

        def zrow(i, carry):
            for g in range(FC // 16):
                zerobuf[i, pl.ds(g * 16, 16)] = z16
            return carry
        lax.fori_loop(0, ZR, zrow, 0)

        def irow(j, carry):
            def ig(g, c2):
                sl = pl.ds(pl.multiple_of(g * 16, 16), 16)
                idxbuf[j, sl] = srcbuf[j, sl] * nchunk + core
                return c2
            return lax.fori_loop(0, 8, ig, carry)
        lax.fori_loop(0, 8, irow, 0)

        def gather_scatter():
            pend = pltpu.async_copy(plflat.at[idxbuf.at[0]],
                                    rowsbuf.at[pl.ds(0, 128)], sem)
            for j in range(8):
                nxt = None
                if j < 7:
                    nxt = pltpu.async_copy(
                        plflat.at[idxbuf.at[j + 1]],
                        rowsbuf.at[pl.ds(((j + 1) % 2) * 128, 128)], sem)
                pend.wait()
                pltpu.sync_copy(rowsbuf.at[pl.ds((j % 2) * 128, 128)],
                                accum.at[dstbuf.at[j]], add=True)
                pend = nxt

        def chunk(k, carry):
            c = core + 2 * k
            _zero_rows(zerobuf, accum, sub * 640, 640)
            plsc.subcore_barrier()
            gather_scatter()
            plsc.subcore_barrier()
            pltpu.sync_copy(accum.at[pl.ds(sub * 640, 640)],
                            agg.at[c, pl.ds(sub * 640, 640)])
            plsc.subcore_barrier()

            def inc_row(j, c1):
                def inc_g(g, c2):
                    sl = pl.ds(pl.multiple_of(g * 16, 16), 16)
                    idxbuf[j, sl] = idxbuf[j, sl] + 2
                    return c2
                return lax.fori_loop(0, 8, inc_g, c1)
            lax.fori_loop(0, 8, inc_row, 0)
            return carry
        lax.fori_loop(0, nchunk_out // 2, chunk, 0)

    return pl.kernel(body, out_type=out_type, mesh=mesh, scratch_types=scratch)


def _mm_body(x_ref, w_ref, b_ref, out_ref, acc_ref):
    k = pl.program_id(2)

    @pl.when(k == 0)
    def _():
        acc_ref[...] = jnp.zeros_like(acc_ref)

    acc_ref[...] += jnp.dot(x_ref[...].astype(jnp.bfloat16),
                            w_ref[...].astype(jnp.bfloat16),
                            preferred_element_type=jnp.float32)

    @pl.when(k == pl.num_programs(2) - 1)
    def _():
        out_ref[...] = (acc_ref[...] + b_ref[...]).astype(out_ref.dtype)


def _matmul(x, w, b2d, Mb, Nb, Kb, out_cols=None, out_dtype=jnp.float32):
    M, K = x.shape
    _, Nc = w.shape
    grid = (M // Mb, Nc // Nb, K // Kb)
    return pl.pallas_call(
        _mm_body,
        grid=grid,
        in_specs=[
            pl.BlockSpec((Mb, Kb), lambda i, j, k: (i, k)),
            pl.BlockSpec((Kb, Nb), lambda i, j, k: (k, j)),
            pl.BlockSpec((1, Nb), lambda i, j, k: (0, j)),
        ],
        out_specs=pl.BlockSpec((Mb, Nb), lambda i, j, k: (i, j)),
        out_shape=jax.ShapeDtypeStruct((M, out_cols or Nc), out_dtype),
        scratch_shapes=[pltpu.VMEM((Mb, Nb), jnp.float32)],
        compiler_params=pltpu.CompilerParams(
            dimension_semantics=("parallel", "parallel", "arbitrary")),
    )(x, w, b2d)


MB1 = 1000


def _h1_body(agg_ref, cntc_ref, p1r_ref, sc_ref, sh_ref, out_ref):
    cnt = cntc_ref[0, :, FC - 1:FC].astype(jnp.float32)
    rc = 1.0 / jnp.maximum(cnt, 1.0)
    z = agg_ref[0].astype(jnp.float32) * rc + p1r_ref[...]
    out_ref[...] = jnp.maximum(z, 0.0) * sc_ref[...] + sh_ref[...]


def _h1_combine(agg1, p1r, scale2d, shift2d):
    grid = (N // MB1, NCHUNK1)
    return pl.pallas_call(
        _h1_body,
        grid=grid,
        in_specs=[
            pl.BlockSpec((1, MB1, FC), lambda i, j: (j, i, 0)),
            pl.BlockSpec((1, MB1, FC), lambda i, j: (NCHUNK1 - 1, i, 0)),
            pl.BlockSpec((MB1, FC), lambda i, j: (i, j)),
            pl.BlockSpec((1, FC), lambda i, j: (0, j)),
            pl.BlockSpec((1, FC), lambda i, j: (0, j)),
        ],
        out_specs=pl.BlockSpec((MB1, FC), lambda i, j: (i, j)),
        out_shape=jax.ShapeDtypeStruct((N, F1P), jnp.float32),
    )(agg1, agg1, p1r, scale2d, shift2d)


MBT = 2000


def _tail_body(agg_ref, p2r_ref, w3, b3, w4, b4, w5, b5,
               out_ref, h2_ref):
    cnt = agg_ref[NCHUNK2 - 1][:, FC - 1:FC].astype(jnp.float32)
    rc = 1.0 / jnp.maximum(cnt, 1.0)
    for c in range(NCHUNK2):
        z = agg_ref[c].astype(jnp.float32) * rc + p2r_ref[:, c * FC:(c + 1) * FC]
        h2_ref[:, c * FC:(c + 1) * FC] = jnp.maximum(z, 0.0)
    a = jnp.maximum(jnp.dot(h2_ref[...], w3[...],
                            preferred_element_type=jnp.float32) + b3[...], 0.0)
    a = jnp.maximum(jnp.dot(a, w4[...],
                            preferred_element_type=jnp.float32) + b4[...], 0.0)
    out_ref[...] = jnp.dot(a, w5[...],
                           preferred_element_type=jnp.float32) + b5[...]


def _tail(agg2, p2r, W3p, b3, W4, b4, W5p, b5p):
    grid = (N // MBT,)
    return pl.pallas_call(
        _tail_body,
        grid=grid,
        in_specs=[
            pl.BlockSpec((NCHUNK2, MBT, FC), lambda i: (0, i, 0)),
            pl.BlockSpec((MBT, F2P), lambda i: (i, 0)),
            pl.BlockSpec(W3p.shape, lambda i: (0, 0)),
            pl.BlockSpec((1, 320), lambda i: (0, 0)),
            pl.BlockSpec(W4.shape, lambda i: (0, 0)),
            pl.BlockSpec((1, 160), lambda i: (0, 0)),
            pl.BlockSpec(W5p.shape, lambda i: (0, 0)),
            pl.BlockSpec((1, 128), lambda i: (0, 0)),
        ],
        out_specs=pl.BlockSpec((MBT, 128), lambda i: (i, 0)),
        out_shape=jax.ShapeDtypeStruct((N, 128), jnp.float32),
        scratch_shapes=[pltpu.VMEM((MBT, F2P), jnp.float32)],
    )(agg2, p2r, W3p, b3, W4, b4, W5p, b5p)


def _forward(x, edge_index, W1_l, W1_r, b1, W2_l, W2_r, b2,
             bn_gamma, bn_beta, bn_mean, bn_var, W3, b3, W4, b4, W5, b5,
             seg1, seg2):
    src = edge_index[0]
    dst = edge_index[1]
    pad = E_PAD - E
    src_p = jnp.concatenate(
        [src, jnp.zeros((pad,), jnp.int32)]).reshape(16, 8, 128)
    dst_p = jnp.concatenate(
        [dst, jnp.full((pad,), N, jnp.int32)]).reshape(16, 8, 128)

    W1l_p = jnp.pad(W1_l, ((0, 0), (0, F1P - H1)))
    W1r_p = jnp.pad(W1_r, ((0, 0), (0, F1P - H1)))
    b1_p = jnp.pad(b1, (0, F1P - H1)).reshape(1, F1P)
    ones_col1 = jnp.zeros((1, F1P), jnp.float32).at[0, F1P - 1].set(1.0)
    P1l = _matmul(x, W1l_p, ones_col1, Mb=1000, Nb=F1P, Kb=1024)
    agg1 = seg1(P1l.reshape(N * NCHUNK1, FC), src_p, dst_p)
    P1r = _matmul(x, W1r_p, b1_p, Mb=1000, Nb=F1P, Kb=1024)

    scale = jnp.pad(bn_gamma * lax.rsqrt(bn_var + 1e-5), (0, F1P - H1))
    shift = jnp.pad(bn_beta - bn_mean * bn_gamma * lax.rsqrt(bn_var + 1e-5),
                    (0, F1P - H1))
    h1 = _h1_combine(agg1, P1r, scale.reshape(1, F1P), shift.reshape(1, F1P))

    W2l_p = jnp.pad(W2_l, ((0, F1P - H1), (0, F1P - H2)))
    W2r_p = jnp.pad(W2_r, ((0, F1P - H1), (0, F2P - H2)))
    b2_p = jnp.pad(b2, (0, F2P - H2)).reshape(1, F2P)
    ones_col2 = jnp.zeros((1, F1P), jnp.float32).at[0, F2P - 1].set(1.0)
    P2l = _matmul(h1, W2l_p, ones_col2, Mb=1000, Nb=F1P, Kb=F1P)
    agg2 = seg2(P2l.reshape(N * NCHUNK1, FC), src_p, dst_p)
    P2r = _matmul(h1, W2r_p, b2_p, Mb=1000, Nb=F2P, Kb=F1P)

    W3p = jnp.pad(W3, ((0, F2P - H2), (0, 0)))
    W5p = jnp.pad(W5, ((0, 0), (0, 126)))
    b5p = jnp.pad(b5, (0, 126)).reshape(1, 128)
    out128 = _tail(agg2, P2r, W3p, b3.reshape(1, 320),
                   W4, b4.reshape(1, 160), W5p, b5p)
    return out128[:, :2]


def kernel(x, edge_index, W1_l, W1_r, b1, W2_l, W2_r, b2,
           bn_gamma, bn_beta, bn_mean, bn_var, W3, b3, W4, b4, W5, b5):
    return _forward(x, edge_index, W1_l, W1_r, b1, W2_l, W2_r, b2,
                    bn_gamma, bn_beta, bn_mean, bn_var, W3, b3, W4, b4, W5, b5,
                    _make_sc_segsum(NCHUNK1, NCHUNK1),
                    _make_sc_segsum(NCHUNK1, NCHUNK2))


# speedup vs baseline: 1.3352x; 1.0589x over previous
import functools

import jax
import jax.numpy as jnp
from jax import lax
from jax.experimental import pallas as pl
from jax.experimental.pallas import tpu as pltpu
from jax.experimental.pallas import tpu_sc as plsc

N = 10000
E = 15625
D_IN = 5120
H1 = 1680
H2 = 640

FC = 128
NCHUNK1 = 14
F1P = NCHUNK1 * FC
NCHUNK2 = 6
F2P = NCHUNK2 * FC
EPT = 1024
E_PAD = 16 * EPT
NROWS = 10240
OUTR = NROWS
ZR = 64


def _zero_rows(zerobuf, accum, r0, total):
    q, off = divmod(total, ZR)
    o = 0
    for _ in range(q):
        pltpu.sync_copy(zerobuf, accum.at[pl.ds(r0 + o, ZR)])
        o += ZR
    if off:
        pltpu.sync_copy(zerobuf.at[pl.ds(0, off)],
                        accum.at[pl.ds(r0 + o, off)])


@functools.lru_cache(maxsize=None)
def _make_sc_segsum(nchunk, nchunk_out):
    mesh = plsc.VectorSubcoreMesh(
        core_axis_name="c", subcore_axis_name="s", num_cores=2, num_subcores=16)

    out_type = jax.ShapeDtypeStruct((nchunk_out, OUTR, FC), jnp.float32)
    scratch = [
        pltpu.VMEM((8, 128), jnp.int32),
        pltpu.VMEM((8, 128), jnp.int32),
        pltpu.VMEM((8, 128), jnp.int32),
        pltpu.VMEM((256, FC), jnp.float32),
        pltpu.VMEM((ZR, FC), jnp.float32),
        pltpu.VMEM_SHARED((NROWS, FC), jnp.float32),
        pltpu.SemaphoreType.DMA,
    ]

    def body(plflat, src3, dst3, agg, *rest):
        (srcbuf, dstbuf, idxbuf, rowsbuf, zerobuf, accum, sem) = rest

        core = lax.axis_index("c")
        sub = lax.axis_index("s")

        pltpu.sync_copy(src3.at[sub], srcbuf)
        pltpu.sync_copy(dst3.at[sub], dstbuf)

        z16 = jnp.zeros((16,), jnp.float32)

        def zrow(i, carry):
            for g in range(FC // 16):
                zerobuf[i, pl.ds(g * 16, 16)] = z16
            return carry
        lax.fori_loop(0, ZR, zrow, 0)

        def irow(j, carry):
            def ig(g, c2):
                sl = pl.ds(pl.multiple_of(g * 16, 16), 16)
                idxbuf[j, sl] = srcbuf[j, sl] * nchunk + core
                return c2
            return lax.fori_loop(0, 8, ig, carry)
        lax.fori_loop(0, 8, irow, 0)

        def gather_scatter():
            pend = pltpu.async_copy(plflat.at[idxbuf.at[0]],
                                    rowsbuf.at[pl.ds(0, 128)], sem)
            for j in range(8):
                nxt = None
                if j < 7:
                    nxt = pltpu.async_copy(
                        plflat.at[idxbuf.at[j + 1]],
                        rowsbuf.at[pl.ds(((j + 1) % 2) * 128, 128)], sem)
                pend.wait()
                pltpu.sync_copy(rowsbuf.at[pl.ds((j % 2) * 128, 128)],
                                accum.at[dstbuf.at[j]], add=True)
                pend = nxt

        def chunk(k, carry):
            c = core + 2 * k
            _zero_rows(zerobuf, accum, sub * 640, 640)
            plsc.subcore_barrier()
            gather_scatter()
            plsc.subcore_barrier()
            pltpu.sync_copy(accum.at[pl.ds(sub * 640, 640)],
                            agg.at[c, pl.ds(sub * 640, 640)])
            plsc.subcore_barrier()

            def inc_row(j, c1):
                def inc_g(g, c2):
                    sl = pl.ds(pl.multiple_of(g * 16, 16), 16)
                    idxbuf[j, sl] = idxbuf[j, sl] + 2
                    return c2
                return lax.fori_loop(0, 8, inc_g, c1)
            lax.fori_loop(0, 8, inc_row, 0)
            return carry
        lax.fori_loop(0, nchunk_out // 2, chunk, 0)

    return pl.kernel(body, out_type=out_type, mesh=mesh, scratch_types=scratch)


def _mm_body(x_ref, w_ref, b_ref, out_ref, acc_ref):
    k = pl.program_id(2)

    @pl.when(k == 0)
    def _():
        acc_ref[...] = jnp.zeros_like(acc_ref)

    acc_ref[...] += jnp.dot(x_ref[...].astype(jnp.bfloat16),
                            w_ref[...].astype(jnp.bfloat16),
                            preferred_element_type=jnp.float32)

    @pl.when(k == pl.num_programs(2) - 1)
    def _():
        out_ref[...] = (acc_ref[...] + b_ref[...]).astype(out_ref.dtype)


def _matmul(x, w, b2d, Mb, Nb, Kb, out_cols=None, out_dtype=jnp.float32):
    M, K = x.shape
    _, Nc = w.shape
    grid = (M // Mb, Nc // Nb, K // Kb)
    return pl.pallas_call(
        _mm_body,
        grid=grid,
        in_specs=[
            pl.BlockSpec((Mb, Kb), lambda i, j, k: (i, k)),
            pl.BlockSpec((Kb, Nb), lambda i, j, k: (k, j)),
            pl.BlockSpec((1, Nb), lambda i, j, k: (0, j)),
        ],
        out_specs=pl.BlockSpec((Mb, Nb), lambda i, j, k: (i, j)),
        out_shape=jax.ShapeDtypeStruct((M, out_cols or Nc), out_dtype),
        scratch_shapes=[pltpu.VMEM((Mb, Nb), jnp.float32)],
        compiler_params=pltpu.CompilerParams(
            dimension_semantics=("parallel", "parallel", "arbitrary")),
    )(x, w, b2d)


MB1 = 1000


def _h1_body(agg_ref, cntc_ref, p1r_ref, sc_ref, sh_ref, out_ref):
    cnt = cntc_ref[0, :, FC - 1:FC].astype(jnp.float32)
    rc = 1.0 / jnp.maximum(cnt, 1.0)
    z = agg_ref[0].astype(jnp.float32) * rc + p1r_ref[...]
    out_ref[...] = jnp.maximum(z, 0.0) * sc_ref[...] + sh_ref[...]


def _h1_combine(agg1, p1r, scale2d, shift2d):
    grid = (N // MB1, NCHUNK1)
    return pl.pallas_call(
        _h1_body,
        grid=grid,
        in_specs=[
            pl.BlockSpec((1, MB1, FC), lambda i, j: (j, i, 0)),
            pl.BlockSpec((1, MB1, FC), lambda i, j: (NCHUNK1 - 1, i, 0)),
            pl.BlockSpec((MB1, FC), lambda i, j: (i, j)),
            pl.BlockSpec((1, FC), lambda i, j: (0, j)),
            pl.BlockSpec((1, FC), lambda i, j: (0, j)),
        ],
        out_specs=pl.BlockSpec((MB1, FC), lambda i, j: (i, j)),
        out_shape=jax.ShapeDtypeStruct((N, F1P), jnp.float32),
    )(agg1, agg1, p1r, scale2d, shift2d)


MBT = 2000


def _tail_body(agg_ref, p2r_ref, w3, b3, w4, b4, w5, b5,
               out_ref, h2_ref):
    cnt = agg_ref[NCHUNK2 - 1][:, FC - 1:FC].astype(jnp.float32)
    rc = 1.0 / jnp.maximum(cnt, 1.0)
    for c in range(NCHUNK2):
        z = agg_ref[c].astype(jnp.float32) * rc + p2r_ref[:, c * FC:(c + 1) * FC]
        h2_ref[:, c * FC:(c + 1) * FC] = jnp.maximum(z, 0.0)
    a = jnp.maximum(jnp.dot(h2_ref[...], w3[...],
                            preferred_element_type=jnp.float32) + b3[...], 0.0)
    a = jnp.maximum(jnp.dot(a, w4[...],
                            preferred_element_type=jnp.float32) + b4[...], 0.0)
    out_ref[...] = jnp.dot(a, w5[...],
                           preferred_element_type=jnp.float32) + b5[...]


def _tail(agg2, p2r, W3p, b3, W4, b4, W5p, b5p):
    grid = (N // MBT,)
    return pl.pallas_call(
        _tail_body,
        grid=grid,
        in_specs=[
            pl.BlockSpec((NCHUNK2, MBT, FC), lambda i: (0, i, 0)),
            pl.BlockSpec((MBT, F2P), lambda i: (i, 0)),
            pl.BlockSpec(W3p.shape, lambda i: (0, 0)),
            pl.BlockSpec((1, 320), lambda i: (0, 0)),
            pl.BlockSpec(W4.shape, lambda i: (0, 0)),
            pl.BlockSpec((1, 160), lambda i: (0, 0)),
            pl.BlockSpec(W5p.shape, lambda i: (0, 0)),
            pl.BlockSpec((1, 128), lambda i: (0, 0)),
        ],
        out_specs=pl.BlockSpec((MBT, 128), lambda i: (i, 0)),
        out_shape=jax.ShapeDtypeStruct((N, 128), jnp.float32),
        scratch_shapes=[pltpu.VMEM((MBT, F2P), jnp.float32)],
    )(agg2, p2r, W3p, b3, W4, b4, W5p, b5p)


def _forward(x, edge_index, W1_l, W1_r, b1, W2_l, W2_r, b2,
             bn_gamma, bn_beta, bn_mean, bn_var, W3, b3, W4, b4, W5, b5,
             seg1, seg2):
    src = edge_index[0]
    dst = edge_index[1]
    pad = E_PAD - E
    src_p = jnp.concatenate(
        [src, jnp.zeros((pad,), jnp.int32)]).reshape(16, 8, 128)
    dst_p = jnp.concatenate(
        [dst, jnp.full((pad,), N, jnp.int32)]).reshape(16, 8, 128)

    W1l_p = jnp.pad(W1_l, ((0, 0), (0, F1P - H1)))
    W1r_p = jnp.pad(W1_r, ((0, 0), (0, F1P - H1)))
    b1_p = jnp.pad(b1, (0, F1P - H1)).reshape(1, F1P)
    ones_col1 = jnp.zeros((1, F1P), jnp.float32).at[0, F1P - 1].set(1.0)
    P1l = _matmul(x, W1l_p, ones_col1, Mb=1000, Nb=F1P, Kb=1024)
    agg1 = seg1(P1l.reshape(N * NCHUNK1, FC), src_p, dst_p)
    P1r = _matmul(x, W1r_p, b1_p, Mb=1000, Nb=F1P, Kb=1024)

    scale = jnp.pad(bn_gamma * lax.rsqrt(bn_var + 1e-5), (0, F1P - H1))
    shift = jnp.pad(bn_beta - bn_mean * bn_gamma * lax.rsqrt(bn_var + 1e-5),
                    (0, F1P - H1))
    h1 = _h1_combine(agg1, P1r, scale.reshape(1, F1P), shift.reshape(1, F1P))

    W2l_p = jnp.pad(W2_l, ((0, F1P - H1), (0, F2P - H2)))
    W2r_p = jnp.pad(W2_r, ((0, F1P - H1), (0, F2P - H2)))
    b2_p = jnp.pad(b2, (0, F2P - H2)).reshape(1, F2P)
    ones_col2 = jnp.zeros((1, F2P), jnp.float32).at[0, F2P - 1].set(1.0)
    P2l = _matmul(h1, W2l_p, ones_col2, Mb=1000, Nb=F2P, Kb=F1P)
    agg2 = seg2(P2l.reshape(N * NCHUNK2, FC), src_p, dst_p)
    P2r = _matmul(h1, W2r_p, b2_p, Mb=1000, Nb=F2P, Kb=F1P)

    W3p = jnp.pad(W3, ((0, F2P - H2), (0, 0)))
    W5p = jnp.pad(W5, ((0, 0), (0, 126)))
    b5p = jnp.pad(b5, (0, 126)).reshape(1, 128)
    out128 = _tail(agg2, P2r, W3p, b3.reshape(1, 320),
                   W4, b4.reshape(1, 160), W5p, b5p)
    return out128[:, :2]


def kernel(x, edge_index, W1_l, W1_r, b1, W2_l, W2_r, b2,
           bn_gamma, bn_beta, bn_mean, bn_var, W3, b3, W4, b4, W5, b5):
    return _forward(x, edge_index, W1_l, W1_r, b1, W2_l, W2_r, b2,
                    bn_gamma, bn_beta, bn_mean, bn_var, W3, b3, W4, b4, W5, b5,
                    _make_sc_segsum(NCHUNK1, NCHUNK1),
                    _make_sc_segsum(NCHUNK2, NCHUNK2))
